# baseline (speedup 1.0000x reference)
import functools

import jax
import jax.numpy as jnp
from jax import lax
from jax.experimental import pallas as pl
from jax.experimental.pallas import tpu as pltpu
from jax.experimental.pallas import tpu_sc as plsc

_B = 4096
_T = 50
_D = 64
_NC, _NS = 2, 16
_NW = _NC * _NS
_BPW = _B // _NW
_TP = _T // 2
_NG = 4
_NI = 8
_STEP = 8
_NFULL = (_T - 2) // _STEP


def _make_sc_gather():
  mesh = plsc.VectorSubcoreMesh(core_axis_name="c", subcore_axis_name="s")

  @functools.partial(
      pl.kernel,
      mesh=mesh,
      out_type=jax.ShapeDtypeStruct((_TP, _B, 2 * _D), jnp.float32),
      compiler_params=pltpu.CompilerParams(use_tc_tiling_on_sc=False),
      scratch_types=(
          [pltpu.VMEM((_BPW,), jnp.int32)] * _NI
          + [pltpu.VMEM((_BPW,), jnp.int32)] * _NI
          + [pltpu.VMEM((_BPW,), jnp.int32)]
          + [pltpu.VMEM((_BPW, _D), jnp.float32)] * _NG
          + [pltpu.SemaphoreType.DMA] * (_NI + 2 * _NG)
      ),
  )
  def sc_gather(ind_hbm, table_hbm, inter_hbm, *scratch):
    ibufs = scratch[:_NI]
    ilists = scratch[_NI:2 * _NI]
    ibase = scratch[2 * _NI]
    gbufs = scratch[2 * _NI + 1:2 * _NI + 1 + _NG]
    isems = scratch[2 * _NI + 1 + _NG:3 * _NI + 1 + _NG]
    gsems = scratch[3 * _NI + 1 + _NG:3 * _NI + 1 + 2 * _NG]
    ssems = scratch[3 * _NI + 1 + 2 * _NG:]
    wid = lax.axis_index("s") * _NC + lax.axis_index("c")
    woff = pl.multiple_of(wid * _BPW, _BPW)
    iota16 = lax.iota(jnp.int32, 16)

    for i in range(_BPW // 16):
      ibase[pl.ds(16 * i, 16)] = iota16 * _T + ((woff + 16 * i) * _T)

    def idx_start(t, j):
      for i in range(_BPW // 16):
        ilists[j][pl.ds(16 * i, 16)] = ibase[pl.ds(16 * i, 16)] + t
      pltpu.make_async_copy(ind_hbm.at[ilists[j]], ibufs[j], isems[j]).start()

    def idx_wait(j):
      pltpu.make_async_copy(ind_hbm.at[ilists[j]], ibufs[j], isems[j]).wait()

    def gather_copy(t, g, j):
      return pltpu.make_async_copy(
          table_hbm.at[ibufs[j]], gbufs[g], gsems[g])

    def store_copy(t, g):
      tp = lax.rem(t, _TP)
      par = t // _TP
      return pltpu.make_async_copy(
          gbufs[g],
          inter_hbm.at[tp, pl.ds(woff, _BPW),
                       pl.ds(pl.multiple_of(par * _D, _D), _D)],
          ssems[g])

    for t0 in range(_NI):
      idx_start(t0, t0)
    for t0 in range(_NG - 1):
      idx_wait(t0)
      gather_copy(t0, t0, t0).start()

    def body(o, carry):
      for k in range(_STEP):
        t = o * _STEP + k
        g = k % _NG
        gn = (k + _NG - 1) % _NG
        jn = (k + _NG - 1) % _NI
        gather_copy(t, g, k).wait()
        if k == 0:
          @pl.when(o >= 1)
          def _(t=t, gn=gn):
            store_copy(t - 1, gn).wait()
        else:
          store_copy(t - 1, gn).wait()

        def fire_next(t=t, gn=gn, jn=jn):
          idx_wait(jn)
          gather_copy(t + _NG - 1, gn, jn).start()

        if k == _STEP - 1:
          pl.when(o < _NFULL - 1)(fire_next)
        else:
          fire_next()
        store_copy(t, g).start()
        if k < 2:
          idx_start(t + _NI, k)
        else:
          @pl.when(o < _NFULL - 1)
          def _(t=t, k=k):
            idx_start(t + _NI, k)
      return carry
    lax.fori_loop(0, _NFULL, body, 0)

    gather_copy(_T - 2, (_T - 2) % _NG, (_T - 2) % _NI).wait()
    store_copy(_T - 3, (_T - 3) % _NG).wait()
    store_copy(_T - 2, (_T - 2) % _NG).start()
    gather_copy(_T - 1, (_T - 1) % _NG, (_T - 1) % _NI).wait()
    store_copy(_T - 1, (_T - 1) % _NG).start()
    store_copy(_T - 2, (_T - 2) % _NG).wait()
    store_copy(_T - 1, (_T - 1) % _NG).wait()

  return sc_gather


_sc_gather = _make_sc_gather()


def _tc_body(x_ref, o_ref):
  xt = x_ref[0].T
  o_ref[0, 0] = xt[:_D, :]
  o_ref[1, 0] = xt[_D:, :]


_tc_transpose = pl.pallas_call(
    _tc_body,
    grid=(_TP,),
    in_specs=[pl.BlockSpec((1, _B, 2 * _D), lambda tp: (tp, 0, 0))],
    out_specs=pl.BlockSpec((2, 1, _D, _B), lambda tp: (0, tp, 0, 0)),
    out_shape=jax.ShapeDtypeStruct((2, _TP, _D, _B), jnp.float32),
)


def kernel(ind, weight):
  inter = _sc_gather(ind.astype(jnp.int32).reshape(-1), weight)
  out4 = _tc_transpose(inter)
  return out4.transpose(3, 0, 1, 2).reshape(_B, _T, _D)


# speedup vs baseline: 1.0281x; 1.0281x over previous
import functools

import jax
import jax.numpy as jnp
from jax import lax
from jax.experimental import pallas as pl
from jax.experimental.pallas import tpu as pltpu
from jax.experimental.pallas import tpu_sc as plsc

_B = 4096
_T = 50
_D = 64
_NC, _NS = 2, 16
_NW = _NC * _NS
_BPW = _B // _NW
_TP = _T // 2
_NG = 4
_NFULL = (_T - 2) // _NG


def _make_sc_gather():
  mesh = plsc.VectorSubcoreMesh(core_axis_name="c", subcore_axis_name="s")

  @functools.partial(
      pl.kernel,
      mesh=mesh,
      out_type=jax.ShapeDtypeStruct((_TP, _B, 2 * _D), jnp.float32),
      compiler_params=pltpu.CompilerParams(use_tc_tiling_on_sc=False),
      scratch_types=(
          [pltpu.VMEM((_T, _BPW), jnp.int32)]
          + [pltpu.VMEM((_BPW, _D), jnp.float32)] * _NG
          + [pltpu.SemaphoreType.DMA] * (2 * _NG)
      ),
  )
  def sc_gather(indt_hbm, table_hbm, inter_hbm, idx_v, *scratch):
    gbufs = scratch[:_NG]
    gsems = scratch[_NG:2 * _NG]
    ssems = scratch[2 * _NG:]
    wid = lax.axis_index("s") * _NC + lax.axis_index("c")
    woff = pl.multiple_of(wid * _BPW, _BPW)

    pltpu.sync_copy(indt_hbm.at[:, pl.ds(woff, _BPW)], idx_v)

    def gather_copy(t, g):
      return pltpu.make_async_copy(
          table_hbm.at[idx_v.at[t]], gbufs[g], gsems[g])

    def store_copy(t, g):
      tp = lax.rem(t, _TP)
      par = t // _TP
      return pltpu.make_async_copy(
          gbufs[g],
          inter_hbm.at[tp, pl.ds(woff, _BPW),
                       pl.ds(pl.multiple_of(par * _D, _D), _D)],
          ssems[g])

    for t0 in range(_NG - 1):
      gather_copy(t0, t0).start()

    def body(o, carry):
      for k in range(_NG):
        t = o * _NG + k
        kn = (k + _NG - 1) % _NG
        gather_copy(t, k).wait()
        if k == 0:
          @pl.when(o >= 1)
          def _(t=t, kn=kn):
            store_copy(t - 1, kn).wait()
        else:
          store_copy(t - 1, kn).wait()
        if k == _NG - 1:
          @pl.when(o < _NFULL - 1)
          def _(t=t, kn=kn):
            gather_copy(t + _NG - 1, kn).start()
        else:
          gather_copy(t + _NG - 1, kn).start()
        store_copy(t, k).start()
      return carry
    lax.fori_loop(0, _NFULL, body, 0)

    gather_copy(_T - 2, (_T - 2) % _NG).wait()
    store_copy(_T - 3, (_T - 3) % _NG).wait()
    store_copy(_T - 2, (_T - 2) % _NG).start()
    gather_copy(_T - 1, (_T - 1) % _NG).wait()
    store_copy(_T - 1, (_T - 1) % _NG).start()
    store_copy(_T - 2, (_T - 2) % _NG).wait()
    store_copy(_T - 1, (_T - 1) % _NG).wait()

  return sc_gather


_sc_gather = _make_sc_gather()


def _tc_body(x_ref, o_ref):
  xt = x_ref[0].T
  o_ref[0, 0] = xt[:_D, :]
  o_ref[1, 0] = xt[_D:, :]


_tc_transpose = pl.pallas_call(
    _tc_body,
    grid=(_TP,),
    in_specs=[pl.BlockSpec((1, _B, 2 * _D), lambda tp: (tp, 0, 0))],
    out_specs=pl.BlockSpec((2, 1, _D, _B), lambda tp: (0, tp, 0, 0)),
    out_shape=jax.ShapeDtypeStruct((2, _TP, _D, _B), jnp.float32),
)


def kernel(ind, weight):
  inter = _sc_gather(ind.astype(jnp.int32).T, weight)
  out4 = _tc_transpose(inter)
  return out4.transpose(3, 0, 1, 2).reshape(_B, _T, _D)
